# baseline (speedup 1.0000x reference)
import functools

import jax
import jax.numpy as jnp
from jax import lax
from jax.experimental import pallas as pl
from jax.experimental.pallas import tpu as pltpu
from jax.experimental.pallas import tpu_sc as plsc

_B = 16384
_V = 100000
_D = 64
_H = 128
_NC = 2
_NS = 16
_NW = _NC * _NS
_BPW = _B // _NW

_CHUNK = 128
_NCH = _BPW // _CHUNK

_XBLK = 8192
_BLK = 2048


def _xform_body(et_ref, w_ref, b_ref, o_ref):
    et = et_ref[...].astype(jnp.bfloat16)
    w = w_ref[...].astype(jnp.bfloat16)
    o_ref[...] = lax.dot_general(et, w, (((0,), (0,)), ((), ())),
                                 preferred_element_type=jnp.float32) + b_ref[...]


def _xform(et, w, b):
    grid = (_V + _XBLK - 1) // _XBLK
    return pl.pallas_call(
        _xform_body,
        grid=(grid,),
        in_specs=[
            pl.BlockSpec((_D, _XBLK), lambda n: (0, n)),
            pl.BlockSpec((_D, _H), lambda n: (0, 0)),
            pl.BlockSpec((1, _H), lambda n: (0, 0)),
        ],
        out_specs=pl.BlockSpec((_XBLK, _H), lambda n: (n, 0)),
        out_shape=jax.ShapeDtypeStruct((_V, _H), jnp.float32),
    )(et, w, b)


def _gather_body(idx_hbm, a_hbm, out_hbm, idx_v, rows_v, sem):
    wid = lax.axis_index("s") * _NC + lax.axis_index("c")
    base = wid * _BPW
    pltpu.sync_copy(idx_hbm.at[pl.ds(wid * _NCH, _NCH)], idx_v)
    copies = []
    for k in range(_NCH):
        copies.append(pltpu.async_copy(
            a_hbm.at[idx_v.at[k]],
            rows_v.at[pl.ds(k * _CHUNK, _CHUNK)], sem))
    for c in copies:
        c.wait()
    pltpu.sync_copy(rows_v, out_hbm.at[pl.ds(base, _BPW)])


@functools.cache
def _gather():
    return pl.kernel(
        _gather_body,
        out_type=jax.ShapeDtypeStruct((_B, _H), jnp.float32),
        mesh=plsc.VectorSubcoreMesh(core_axis_name="c", subcore_axis_name="s",
                                    num_cores=_NC, num_subcores=_NS),
        scratch_types=[
            pltpu.VMEM((_NCH, _CHUNK), jnp.int32),
            pltpu.VMEM((_BPW, _H), jnp.float32),
            pltpu.SemaphoreType.DMA,
        ],
    )


def _tail_body(au_ref, ai_ref, w2_ref, b2_ref, w3t_ref, b3_ref, o_ref):
    h1 = jnp.maximum(au_ref[...] + ai_ref[...], 0.0)
    h2 = jnp.dot(h1.astype(jnp.bfloat16), w2_ref[...],
                 preferred_element_type=jnp.float32)
    h2 = jnp.maximum(h2 + b2_ref[...], 0.0)
    logit = jnp.sum(h2 * w3t_ref[...], axis=1, keepdims=True) + b3_ref[...]
    o_ref[...] = 1.0 / (1.0 + jnp.exp(-logit))


def _tail(au, ai, w2, b2, w3t, b3):
    full = lambda s: pl.BlockSpec(s, lambda n: (0, 0))
    return pl.pallas_call(
        _tail_body,
        grid=(_B // _BLK,),
        in_specs=[
            pl.BlockSpec((_BLK, _H), lambda n: (n, 0)),
            pl.BlockSpec((_BLK, _H), lambda n: (n, 0)),
            full((_H, _D)),
            full((1, _D)),
            full((1, _D)),
            full((1, 1)),
        ],
        out_specs=pl.BlockSpec((_BLK, 1), lambda n: (n, 0)),
        out_shape=jax.ShapeDtypeStruct((_B, 1), jnp.float32),
    )(au, ai, w2, b2, w3t, b3)


def kernel(inputs, user_emb, item_emb, W1, b1, W2, b2, W3, b3):
    uidx = inputs[:, 0].reshape(_B // _CHUNK, _CHUNK)
    iidx = inputs[:, 1].reshape(_B // _CHUNK, _CHUNK)
    zeros = jnp.zeros((1, _H), jnp.float32)
    g = _gather()
    au = _xform(user_emb.T, W1[:_D], b1.reshape(1, _H))
    au_g = g(uidx, au)
    ai = _xform(item_emb.T, W1[_D:], zeros)
    ai_g = g(iidx, ai)
    return _tail(au_g, ai_g, W2.astype(jnp.bfloat16), b2.reshape(1, _D),
                 W3.reshape(1, _D), b3.reshape(1, 1))


# speedup vs baseline: 1.1132x; 1.1132x over previous
import functools

import jax
import jax.numpy as jnp
from jax import lax
from jax.experimental import pallas as pl
from jax.experimental.pallas import tpu as pltpu
from jax.experimental.pallas import tpu_sc as plsc

_B = 16384
_V = 100000
_D = 64
_H = 128
_NC = 2
_NS = 16
_NW = _NC * _NS
_BPW = _B // _NW

_CHUNK = 128
_NCH = _BPW // _CHUNK

_XBLK = 8192
_BLK = 2048


def _xform_body(et_ref, w_ref, b_ref, o_ref):
    et = et_ref[...].astype(jnp.bfloat16)
    w = w_ref[...].astype(jnp.bfloat16)
    o_ref[...] = lax.dot_general(et, w, (((0,), (0,)), ((), ())),
                                 preferred_element_type=jnp.float32) + b_ref[...]


def _xform(et, w, b):
    grid = (_V + _XBLK - 1) // _XBLK
    return pl.pallas_call(
        _xform_body,
        grid=(grid,),
        in_specs=[
            pl.BlockSpec((_D, _XBLK), lambda n: (0, n)),
            pl.BlockSpec((_D, _H), lambda n: (0, 0)),
            pl.BlockSpec((1, _H), lambda n: (0, 0)),
        ],
        out_specs=pl.BlockSpec((_XBLK, _H), lambda n: (n, 0)),
        out_shape=jax.ShapeDtypeStruct((_V, _H), jnp.float32),
    )(et, w, b)


def _gather_body(uidx_hbm, iidx_hbm, au_hbm, ai_hbm, out_hbm,
                 uidx_v, iidx_v, rows_v, sem):
    wid = lax.axis_index("s") * _NC + lax.axis_index("c")
    base = wid * _BPW
    pltpu.sync_copy(uidx_hbm.at[pl.ds(wid * _NCH, _NCH)], uidx_v)
    pltpu.sync_copy(iidx_hbm.at[pl.ds(wid * _NCH, _NCH)], iidx_v)
    copies = []
    for k in range(_NCH):
        copies.append(pltpu.async_copy(
            au_hbm.at[uidx_v.at[k]],
            rows_v.at[pl.ds(k * _CHUNK, _CHUNK)], sem))
    for c in copies:
        c.wait()
    copies = []
    for k in range(_NCH):
        copies.append(pltpu.async_copy(
            ai_hbm.at[iidx_v.at[k]],
            rows_v.at[pl.ds(k * _CHUNK, _CHUNK)], sem, add=True))
    for c in copies:
        c.wait()
    pltpu.sync_copy(rows_v, out_hbm.at[pl.ds(base, _BPW)])


@functools.cache
def _gather():
    return pl.kernel(
        _gather_body,
        out_type=jax.ShapeDtypeStruct((_B, _H), jnp.float32),
        mesh=plsc.VectorSubcoreMesh(core_axis_name="c", subcore_axis_name="s",
                                    num_cores=_NC, num_subcores=_NS),
        scratch_types=[
            pltpu.VMEM((_NCH, _CHUNK), jnp.int32),
            pltpu.VMEM((_NCH, _CHUNK), jnp.int32),
            pltpu.VMEM((_BPW, _H), jnp.float32),
            pltpu.SemaphoreType.DMA,
        ],
    )


def _tail_body(a_ref, w2_ref, b2_ref, w3t_ref, b3_ref, o_ref):
    h1 = jnp.maximum(a_ref[...], 0.0)
    h2 = jnp.dot(h1.astype(jnp.bfloat16), w2_ref[...],
                 preferred_element_type=jnp.float32)
    h2 = jnp.maximum(h2 + b2_ref[...], 0.0)
    logit = lax.dot_general(w3t_ref[...], h2, (((1,), (1,)), ((), ())),
                            preferred_element_type=jnp.float32) + b3_ref[...]
    o_ref[...] = 1.0 / (1.0 + jnp.exp(-logit))


def _tail(a, w2, b2, w3t, b3):
    full = lambda s: pl.BlockSpec(s, lambda n: (0, 0))
    return pl.pallas_call(
        _tail_body,
        grid=(_B // _BLK,),
        in_specs=[
            pl.BlockSpec((_BLK, _H), lambda n: (n, 0)),
            full((_H, _D)),
            full((1, _D)),
            full((1, _D)),
            full((1, 1)),
        ],
        out_specs=pl.BlockSpec((1, _BLK), lambda n: (0, n)),
        out_shape=jax.ShapeDtypeStruct((1, _B), jnp.float32),
    )(a, w2, b2, w3t, b3)


def kernel(inputs, user_emb, item_emb, W1, b1, W2, b2, W3, b3):
    idx_t = inputs.T
    uidx = idx_t[0].reshape(_B // _CHUNK, _CHUNK)
    iidx = idx_t[1].reshape(_B // _CHUNK, _CHUNK)
    zeros = jnp.zeros((1, _H), jnp.float32)
    au = _xform(user_emb.T, W1[:_D], b1.reshape(1, _H))
    ai = _xform(item_emb.T, W1[_D:], zeros)
    a_g = _gather()(uidx, iidx, au, ai)
    out_t = _tail(a_g, W2.astype(jnp.bfloat16), b2.reshape(1, _D),
                  W3.reshape(1, _D), b3.reshape(1, 1))
    return out_t.T


# speedup vs baseline: 1.1399x; 1.0239x over previous
import functools

import jax
import jax.numpy as jnp
from jax import lax
from jax.experimental import pallas as pl
from jax.experimental.pallas import tpu as pltpu
from jax.experimental.pallas import tpu_sc as plsc

_B = 16384
_V = 100000
_D = 64
_H = 128
_NC = 2
_NS = 16
_NW = _NC * _NS
_BPW = _B // _NW

_CHUNK = 128
_NCH = _BPW // _CHUNK

_XBLK = 12800
_BLK = 2048


def _xform_body(et_ref, w_ref, b_ref, o_ref):
    et = et_ref[...].astype(jnp.bfloat16)
    w = w_ref[...].astype(jnp.bfloat16)
    o_ref[...] = lax.dot_general(et, w, (((0,), (0,)), ((), ())),
                                 preferred_element_type=jnp.float32) + b_ref[...]


def _xform(et, w, b):
    grid = (_V + _XBLK - 1) // _XBLK
    return pl.pallas_call(
        _xform_body,
        grid=(grid,),
        in_specs=[
            pl.BlockSpec((_D, _XBLK), lambda n: (0, n)),
            pl.BlockSpec((_D, _H), lambda n: (0, 0)),
            pl.BlockSpec((1, _H), lambda n: (0, 0)),
        ],
        out_specs=pl.BlockSpec((_XBLK, _H), lambda n: (n, 0)),
        out_shape=jax.ShapeDtypeStruct((_V, _H), jnp.float32),
    )(et, w, b)


def _gather_body(uidx_hbm, iidx_hbm, au_hbm, ai_hbm, out_hbm,
                 uidx_v, iidx_v, rows_v, sem):
    wid = lax.axis_index("s") * _NC + lax.axis_index("c")
    base = wid * _BPW
    pltpu.sync_copy(uidx_hbm.at[pl.ds(wid * _NCH, _NCH)], uidx_v)
    pltpu.sync_copy(iidx_hbm.at[pl.ds(wid * _NCH, _NCH)], iidx_v)
    copies = []
    for k in range(_NCH):
        copies.append(pltpu.async_copy(
            au_hbm.at[uidx_v.at[k]],
            rows_v.at[pl.ds(k * _CHUNK, _CHUNK)], sem))
    for c in copies:
        c.wait()
    copies = []
    for k in range(_NCH):
        copies.append(pltpu.async_copy(
            ai_hbm.at[iidx_v.at[k]],
            rows_v.at[pl.ds(k * _CHUNK, _CHUNK)], sem, add=True))
    for c in copies:
        c.wait()
    pltpu.sync_copy(rows_v, out_hbm.at[pl.ds(base, _BPW)])


@functools.cache
def _gather():
    return pl.kernel(
        _gather_body,
        out_type=jax.ShapeDtypeStruct((_B, _H), jnp.float32),
        mesh=plsc.VectorSubcoreMesh(core_axis_name="c", subcore_axis_name="s",
                                    num_cores=_NC, num_subcores=_NS),
        scratch_types=[
            pltpu.VMEM((_NCH, _CHUNK), jnp.int32),
            pltpu.VMEM((_NCH, _CHUNK), jnp.int32),
            pltpu.VMEM((_BPW, _H), jnp.float32),
            pltpu.SemaphoreType.DMA,
        ],
    )


def _tail_body(a_ref, w2_ref, b2_ref, w3t_ref, b3_ref, o_ref):
    h1 = jnp.maximum(a_ref[...], 0.0)
    h2 = jnp.dot(h1.astype(jnp.bfloat16), w2_ref[...],
                 preferred_element_type=jnp.float32)
    h2 = jnp.maximum(h2 + b2_ref[...], 0.0)
    logit = lax.dot_general(w3t_ref[...], h2, (((1,), (1,)), ((), ())),
                            preferred_element_type=jnp.float32) + b3_ref[...]
    o_ref[...] = 1.0 / (1.0 + jnp.exp(-logit))


def _tail(a, w2, b2, w3t, b3):
    full = lambda s: pl.BlockSpec(s, lambda n: (0, 0))
    return pl.pallas_call(
        _tail_body,
        grid=(_B // _BLK,),
        in_specs=[
            pl.BlockSpec((_BLK, _H), lambda n: (n, 0)),
            full((_H, _D)),
            full((1, _D)),
            full((1, _D)),
            full((1, 1)),
        ],
        out_specs=pl.BlockSpec((1, _BLK), lambda n: (0, n)),
        out_shape=jax.ShapeDtypeStruct((1, _B), jnp.float32),
    )(a, w2, b2, w3t, b3)


def kernel(inputs, user_emb, item_emb, W1, b1, W2, b2, W3, b3):
    idx_t = inputs.T
    uidx = idx_t[0].reshape(_B // _CHUNK, _CHUNK)
    iidx = idx_t[1].reshape(_B // _CHUNK, _CHUNK)
    zeros = jnp.zeros((1, _H), jnp.float32)
    au = _xform(user_emb.T, W1[:_D], b1.reshape(1, _H))
    ai = _xform(item_emb.T, W1[_D:], zeros)
    a_g = _gather()(uidx, iidx, au, ai)
    out_t = _tail(a_g, W2.astype(jnp.bfloat16), b2.reshape(1, _D),
                  W3.reshape(1, _D), b3.reshape(1, 1))
    return out_t.T


# speedup vs baseline: 1.1789x; 1.0343x over previous
import functools

import jax
import jax.numpy as jnp
from jax import lax
from jax.experimental import pallas as pl
from jax.experimental.pallas import tpu as pltpu
from jax.experimental.pallas import tpu_sc as plsc

_B = 16384
_V = 100000
_D = 64
_H = 128
_NC = 2
_NS = 16
_NW = _NC * _NS
_BPW = _B // _NW

_CHUNK = 128
_NCH = _BPW // _CHUNK

_XBLK = 20480
_BLK = 2048


def _xform_body(et_ref, w_ref, b_ref, o_ref):
    et = et_ref[...].astype(jnp.bfloat16)
    w = w_ref[...].astype(jnp.bfloat16)
    o_ref[...] = lax.dot_general(et, w, (((0,), (0,)), ((), ())),
                                 preferred_element_type=jnp.float32) + b_ref[...]


def _xform(et, w, b):
    grid = (_V + _XBLK - 1) // _XBLK
    return pl.pallas_call(
        _xform_body,
        grid=(grid,),
        in_specs=[
            pl.BlockSpec((_D, _XBLK), lambda n: (0, n)),
            pl.BlockSpec((_D, _H), lambda n: (0, 0)),
            pl.BlockSpec((1, _H), lambda n: (0, 0)),
        ],
        out_specs=pl.BlockSpec((_XBLK, _H), lambda n: (n, 0)),
        out_shape=jax.ShapeDtypeStruct((_V, _H), jnp.float32),
    )(et, w, b)


def _gather_body(idx_hbm, au_hbm, ai_hbm, out_hbm,
                 uidx_v, iidx_v, rows_v, sem):
    wid = lax.axis_index("s") * _NC + lax.axis_index("c")
    base = wid * _BPW
    pltpu.sync_copy(idx_hbm.at[0, pl.ds(wid * _NCH, _NCH)], uidx_v)
    pltpu.sync_copy(idx_hbm.at[1, pl.ds(wid * _NCH, _NCH)], iidx_v)
    copies = []
    for k in range(_NCH):
        copies.append(pltpu.async_copy(
            au_hbm.at[uidx_v.at[k]],
            rows_v.at[pl.ds(k * _CHUNK, _CHUNK)], sem))
    for c in copies:
        c.wait()
    copies = []
    for k in range(_NCH):
        copies.append(pltpu.async_copy(
            ai_hbm.at[iidx_v.at[k]],
            rows_v.at[pl.ds(k * _CHUNK, _CHUNK)], sem, add=True))
    for c in copies:
        c.wait()
    pltpu.sync_copy(rows_v, out_hbm.at[pl.ds(base, _BPW)])


@functools.cache
def _gather():
    return pl.kernel(
        _gather_body,
        out_type=jax.ShapeDtypeStruct((_B, _H), jnp.float32),
        mesh=plsc.VectorSubcoreMesh(core_axis_name="c", subcore_axis_name="s",
                                    num_cores=_NC, num_subcores=_NS),
        scratch_types=[
            pltpu.VMEM((_NCH, _CHUNK), jnp.int32),
            pltpu.VMEM((_NCH, _CHUNK), jnp.int32),
            pltpu.VMEM((_BPW, _H), jnp.float32),
            pltpu.SemaphoreType.DMA,
        ],
    )


def _tail_body(a_ref, w2_ref, b2_ref, w3t_ref, b3_ref, o_ref):
    h1 = jnp.maximum(a_ref[...], 0.0)
    h2 = jnp.dot(h1.astype(jnp.bfloat16), w2_ref[...],
                 preferred_element_type=jnp.float32)
    h2 = jnp.maximum(h2 + b2_ref[...], 0.0)
    logit = lax.dot_general(w3t_ref[...], h2, (((1,), (1,)), ((), ())),
                            preferred_element_type=jnp.float32) + b3_ref[...]
    o_ref[...] = 1.0 / (1.0 + jnp.exp(-logit))


def _tail(a, w2, b2, w3t, b3):
    full = lambda s: pl.BlockSpec(s, lambda n: (0, 0))
    return pl.pallas_call(
        _tail_body,
        grid=(_B // _BLK,),
        in_specs=[
            pl.BlockSpec((_BLK, _H), lambda n: (n, 0)),
            full((_H, _D)),
            full((1, _D)),
            full((1, _D)),
            full((1, 1)),
        ],
        out_specs=pl.BlockSpec((1, _BLK), lambda n: (0, n)),
        out_shape=jax.ShapeDtypeStruct((1, _B), jnp.float32),
    )(a, w2, b2, w3t, b3)


def kernel(inputs, user_emb, item_emb, W1, b1, W2, b2, W3, b3):
    idx3 = inputs.T.reshape(2, _B // _CHUNK, _CHUNK)
    zeros = jnp.zeros((1, _H), jnp.float32)
    au = _xform(user_emb.T, W1[:_D], b1.reshape(1, _H))
    ai = _xform(item_emb.T, W1[_D:], zeros)
    a_g = _gather()(idx3, au, ai)
    out_t = _tail(a_g, W2.astype(jnp.bfloat16), b2.reshape(1, _D),
                  W3.reshape(1, _D), b3.reshape(1, 1))
    return out_t.T


# speedup vs baseline: 1.1885x; 1.0081x over previous
import functools

import jax
import jax.numpy as jnp
from jax import lax
from jax.experimental import pallas as pl
from jax.experimental.pallas import tpu as pltpu
from jax.experimental.pallas import tpu_sc as plsc

_B = 16384
_V = 100000
_D = 64
_H = 128
_NC = 2
_NS = 16
_NW = _NC * _NS
_BPW = _B // _NW

_CHUNK = 128
_NCH = _BPW // _CHUNK

_XBLK = 25600
_BLK = 2048


def _xform_body(et_ref, w_ref, b_ref, o_ref):
    et = et_ref[...].astype(jnp.bfloat16)
    w = w_ref[...].astype(jnp.bfloat16)
    o_ref[...] = lax.dot_general(et, w, (((0,), (0,)), ((), ())),
                                 preferred_element_type=jnp.float32) + b_ref[...]


def _xform(et, w, b):
    grid = (_V + _XBLK - 1) // _XBLK
    return pl.pallas_call(
        _xform_body,
        grid=(grid,),
        in_specs=[
            pl.BlockSpec((_D, _XBLK), lambda n: (0, n)),
            pl.BlockSpec((_D, _H), lambda n: (0, 0)),
            pl.BlockSpec((1, _H), lambda n: (0, 0)),
        ],
        out_specs=pl.BlockSpec((_XBLK, _H), lambda n: (n, 0)),
        out_shape=jax.ShapeDtypeStruct((_V, _H), jnp.float32),
    )(et, w, b)


def _gather_body(idx_hbm, au_hbm, ai_hbm, out_hbm,
                 uidx_v, iidx_v, rows_v, sem):
    wid = lax.axis_index("s") * _NC + lax.axis_index("c")
    base = wid * _BPW
    pltpu.sync_copy(idx_hbm.at[0, pl.ds(wid * _NCH, _NCH)], uidx_v)
    pltpu.sync_copy(idx_hbm.at[1, pl.ds(wid * _NCH, _NCH)], iidx_v)
    copies = []
    for k in range(_NCH):
        copies.append(pltpu.async_copy(
            au_hbm.at[uidx_v.at[k]],
            rows_v.at[pl.ds(k * _CHUNK, _CHUNK)], sem))
    for c in copies:
        c.wait()
    copies = []
    for k in range(_NCH):
        copies.append(pltpu.async_copy(
            ai_hbm.at[iidx_v.at[k]],
            rows_v.at[pl.ds(k * _CHUNK, _CHUNK)], sem, add=True))
    for c in copies:
        c.wait()
    pltpu.sync_copy(rows_v, out_hbm.at[pl.ds(base, _BPW)])


@functools.cache
def _gather():
    return pl.kernel(
        _gather_body,
        out_type=jax.ShapeDtypeStruct((_B, _H), jnp.float32),
        mesh=plsc.VectorSubcoreMesh(core_axis_name="c", subcore_axis_name="s",
                                    num_cores=_NC, num_subcores=_NS),
        scratch_types=[
            pltpu.VMEM((_NCH, _CHUNK), jnp.int32),
            pltpu.VMEM((_NCH, _CHUNK), jnp.int32),
            pltpu.VMEM((_BPW, _H), jnp.float32),
            pltpu.SemaphoreType.DMA,
        ],
    )


def _tail_body(a_ref, w2_ref, b2_ref, w3t_ref, b3_ref, o_ref):
    h1 = jnp.maximum(a_ref[...], 0.0)
    h2 = jnp.dot(h1.astype(jnp.bfloat16), w2_ref[...],
                 preferred_element_type=jnp.float32)
    h2 = jnp.maximum(h2 + b2_ref[...], 0.0)
    logit = lax.dot_general(w3t_ref[...], h2, (((1,), (1,)), ((), ())),
                            preferred_element_type=jnp.float32) + b3_ref[...]
    o_ref[...] = 1.0 / (1.0 + jnp.exp(-logit))


def _tail(a, w2, b2, w3t, b3):
    full = lambda s: pl.BlockSpec(s, lambda n: (0, 0))
    return pl.pallas_call(
        _tail_body,
        grid=(_B // _BLK,),
        in_specs=[
            pl.BlockSpec((_BLK, _H), lambda n: (n, 0)),
            full((_H, _D)),
            full((1, _D)),
            full((1, _D)),
            full((1, 1)),
        ],
        out_specs=pl.BlockSpec((1, _BLK), lambda n: (0, n)),
        out_shape=jax.ShapeDtypeStruct((1, _B), jnp.float32),
    )(a, w2, b2, w3t, b3)


def kernel(inputs, user_emb, item_emb, W1, b1, W2, b2, W3, b3):
    idx3 = inputs.T.reshape(2, _B // _CHUNK, _CHUNK)
    zeros = jnp.zeros((1, _H), jnp.float32)
    au = _xform(user_emb.T, W1[:_D], b1.reshape(1, _H))
    ai = _xform(item_emb.T, W1[_D:], zeros)
    a_g = _gather()(idx3, au, ai)
    out_t = _tail(a_g, W2.astype(jnp.bfloat16), b2.reshape(1, _D),
                  W3.reshape(1, _D), b3.reshape(1, 1))
    return out_t.T
